# Initial kernel scaffold; baseline (speedup 1.0000x reference)
import jax
import jax.numpy as jnp
from jax.experimental import pallas as pl


def kernel(efeat, nfeat, edge_index, edge_w1, edge_b1, edge_w2, edge_b2, edge_ln_s, edge_ln_b, node_w1, node_b1, node_w2, node_b2, node_ln_s, node_ln_b):
    raise NotImplementedError("write your pallas kernel here")


# speedup vs baseline: 1.1081x; 1.1081x over previous
import functools

import jax
import jax.numpy as jnp
from jax.experimental import pallas as pl

EPS = 1e-5


def _proj_body(nf_ref, w1b_ref, w1c_ref, b1_ref, p_ref, q_ref):
    nf = nf_ref[...]
    p_ref[...] = jnp.dot(nf, w1b_ref[...], preferred_element_type=jnp.float32) + b1_ref[...]
    q_ref[...] = jnp.dot(nf, w1c_ref[...], preferred_element_type=jnp.float32)


def _edge_body(e_ref, gs_ref, gd_ref, w1a_ref, w2_ref, b2_ref, s_ref, b_ref, out_ref):
    e = e_ref[...]
    z = jnp.dot(e, w1a_ref[...], preferred_element_type=jnp.float32)
    z = z + gs_ref[...] + gd_ref[...]
    h = z * jax.nn.sigmoid(z)
    m = jnp.dot(h, w2_ref[...], preferred_element_type=jnp.float32) + b2_ref[...]
    mu = jnp.mean(m, axis=-1, keepdims=True)
    c = m - mu
    var = jnp.mean(c * c, axis=-1, keepdims=True)
    out_ref[...] = e + c * jax.lax.rsqrt(var + EPS) * s_ref[...] + b_ref[...]


def _node_body(agg_ref, nf_ref, w1a_ref, w1b_ref, b1_ref, w2_ref, b2_ref, s_ref,
               b_ref, out_ref):
    nf = nf_ref[...]
    z = jnp.dot(agg_ref[...], w1a_ref[...], preferred_element_type=jnp.float32)
    z = z + jnp.dot(nf, w1b_ref[...], preferred_element_type=jnp.float32) + b1_ref[...]
    h = z * jax.nn.sigmoid(z)
    m = jnp.dot(h, w2_ref[...], preferred_element_type=jnp.float32) + b2_ref[...]
    mu = jnp.mean(m, axis=-1, keepdims=True)
    c = m - mu
    var = jnp.mean(c * c, axis=-1, keepdims=True)
    out_ref[...] = nf + c * jax.lax.rsqrt(var + EPS) * s_ref[...] + b_ref[...]


def _row_spec(tile, d):
    return pl.BlockSpec((tile, d), lambda i: (i, 0))


def _full_spec(shape):
    return pl.BlockSpec(shape, lambda i: tuple(0 for _ in shape))


def _pick_tile(n, want):
    t = min(want, n)
    while n % t:
        t -= 1
    return t


def kernel(efeat, nfeat, edge_index, edge_w1, edge_b1, edge_w2, edge_b2,
           edge_ln_s, edge_ln_b, node_w1, node_b1, node_w2, node_b2,
           node_ln_s, node_ln_b):
    E, D = efeat.shape
    N, _ = nfeat.shape
    L = edge_w1.shape[0]
    H = edge_w1.shape[2]

    tile_e = _pick_tile(E, 1000)
    tile_n = _pick_tile(N, 1000)

    src = edge_index[0]
    dst = edge_index[1]

    proj_call = pl.pallas_call(
        _proj_body,
        grid=(N // tile_n,),
        in_specs=[_row_spec(tile_n, D), _full_spec((D, H)), _full_spec((D, H)),
                  _full_spec((1, H))],
        out_specs=[_row_spec(tile_n, H), _row_spec(tile_n, H)],
        out_shape=[jax.ShapeDtypeStruct((N, H), jnp.float32),
                   jax.ShapeDtypeStruct((N, H), jnp.float32)],
    )

    edge_call = pl.pallas_call(
        _edge_body,
        grid=(E // tile_e,),
        in_specs=[_row_spec(tile_e, D), _row_spec(tile_e, H), _row_spec(tile_e, H),
                  _full_spec((D, H)), _full_spec((H, D)), _full_spec((1, D)),
                  _full_spec((1, D)), _full_spec((1, D))],
        out_specs=_row_spec(tile_e, D),
        out_shape=jax.ShapeDtypeStruct((E, D), jnp.float32),
    )

    node_call = pl.pallas_call(
        _node_body,
        grid=(N // tile_n,),
        in_specs=[_row_spec(tile_n, D), _row_spec(tile_n, D),
                  _full_spec((D, H)), _full_spec((D, H)), _full_spec((1, H)),
                  _full_spec((H, D)), _full_spec((1, D)), _full_spec((1, D)),
                  _full_spec((1, D))],
        out_specs=_row_spec(tile_n, D),
        out_shape=jax.ShapeDtypeStruct((N, D), jnp.float32),
    )

    for l in range(L):
        w1 = edge_w1[l]
        p, q = proj_call(nfeat, w1[D:2 * D], w1[2 * D:], edge_b1[l][None, :])
        gs = p[src]
        gd = q[dst]
        efeat = edge_call(efeat, gs, gd, w1[:D], edge_w2[l],
                          edge_b2[l][None, :], edge_ln_s[l][None, :],
                          edge_ln_b[l][None, :])
        agg = jax.ops.segment_sum(efeat, dst, num_segments=N)
        nw1 = node_w1[l]
        nfeat = node_call(agg, nfeat, nw1[:D], nw1[D:], node_b1[l][None, :],
                          node_w2[l], node_b2[l][None, :],
                          node_ln_s[l][None, :], node_ln_b[l][None, :])
    return (efeat, nfeat)


# speedup vs baseline: 2.9945x; 2.7023x over previous
import functools

import jax
import jax.numpy as jnp
from jax import lax
from jax.experimental import pallas as pl
from jax.experimental.pallas import tpu as pltpu
from jax.experimental.pallas import tpu_sc as plsc

EPS = 1e-5


def _proj_body(nf_ref, w1b_ref, w1c_ref, b1_ref, p_ref, q_ref):
    nf = nf_ref[...]
    p_ref[...] = jnp.dot(nf, w1b_ref[...], preferred_element_type=jnp.float32) + b1_ref[...]
    q_ref[...] = jnp.dot(nf, w1c_ref[...], preferred_element_type=jnp.float32)


def _edge_body(e_ref, gs_ref, gd_ref, w1a_ref, w2_ref, b2_ref, s_ref, b_ref, out_ref):
    e = e_ref[...]
    z = jnp.dot(e, w1a_ref[...], preferred_element_type=jnp.float32)
    z = z + gs_ref[...] + gd_ref[...]
    h = z * jax.nn.sigmoid(z)
    m = jnp.dot(h, w2_ref[...], preferred_element_type=jnp.float32) + b2_ref[...]
    mu = jnp.mean(m, axis=-1, keepdims=True)
    c = m - mu
    var = jnp.mean(c * c, axis=-1, keepdims=True)
    out_ref[...] = e + c * jax.lax.rsqrt(var + EPS) * s_ref[...] + b_ref[...]


def _node_body(a0_ref, a1_ref, nf_ref, w1a_ref, w1b_ref, b1_ref, w2_ref, b2_ref,
               s_ref, b_ref, out_ref):
    nf = nf_ref[...]
    agg = a0_ref[...] + a1_ref[...]
    z = jnp.dot(agg, w1a_ref[...], preferred_element_type=jnp.float32)
    z = z + jnp.dot(nf, w1b_ref[...], preferred_element_type=jnp.float32) + b1_ref[...]
    h = z * jax.nn.sigmoid(z)
    m = jnp.dot(h, w2_ref[...], preferred_element_type=jnp.float32) + b2_ref[...]
    mu = jnp.mean(m, axis=-1, keepdims=True)
    c = m - mu
    var = jnp.mean(c * c, axis=-1, keepdims=True)
    out_ref[...] = nf + c * jax.lax.rsqrt(var + EPS) * s_ref[...] + b_ref[...]


def _row_spec(tile, d):
    return pl.BlockSpec((tile, d), lambda i: (i, 0))


def _full_spec(shape):
    return pl.BlockSpec(shape, lambda i: tuple(0 for _ in shape))


def _pick_tile(n, want):
    t = min(want, n)
    while n % t:
        t -= 1
    return t


_SC_MESH = plsc.VectorSubcoreMesh(core_axis_name="c", subcore_axis_name="s")
_NC, _NS = 2, 16
_NW = _NC * _NS


def _make_gather(E, N, H, chunk):
    n_chunks = E // (_NW * chunk)

    @functools.partial(
        pl.kernel,
        out_type=[jax.ShapeDtypeStruct((E, H), jnp.float32),
                  jax.ShapeDtypeStruct((E, H), jnp.float32)],
        mesh=_SC_MESH,
        scratch_types=[
            pltpu.VMEM((chunk,), jnp.int32),
            pltpu.VMEM((chunk,), jnp.int32),
            pltpu.VMEM((chunk, H), jnp.float32),
            pltpu.VMEM((chunk, H), jnp.float32),
            pltpu.SemaphoreType.DMA,
            pltpu.SemaphoreType.DMA,
        ],
    )
    def gather_kernel(p_hbm, q_hbm, src_hbm, dst_hbm, gs_hbm, gd_hbm,
                      idx_s, idx_d, rows_s, rows_d, sem_s, sem_d):
        wid = lax.axis_index("s") * _NC + lax.axis_index("c")
        for ci in range(n_chunks):
            base = (wid * n_chunks + ci) * chunk
            pltpu.sync_copy(src_hbm.at[pl.ds(base, chunk)], idx_s)
            pltpu.sync_copy(dst_hbm.at[pl.ds(base, chunk)], idx_d)
            cp_s = pltpu.async_copy(p_hbm.at[idx_s], rows_s, sem_s)
            cp_d = pltpu.async_copy(q_hbm.at[idx_d], rows_d, sem_d)
            cp_s.wait()
            cp_d.wait()
            pltpu.sync_copy(rows_s, gs_hbm.at[pl.ds(base, chunk)])
            pltpu.sync_copy(rows_d, gd_hbm.at[pl.ds(base, chunk)])

    return gather_kernel


def _make_scatter(E, N_pad, D, chunk):
    e_per_core = E // _NC
    e_per_sub = e_per_core // _NS
    n_chunks = e_per_sub // chunk
    rows_per_sub = N_pad // _NS

    @functools.partial(
        pl.kernel,
        out_type=jax.ShapeDtypeStruct((_NC * N_pad, D), jnp.float32),
        mesh=_SC_MESH,
        scratch_types=[
            pltpu.VMEM((chunk,), jnp.int32),
            pltpu.VMEM((chunk, D), jnp.float32),
            pltpu.VMEM_SHARED((N_pad, D), jnp.float32),
        ],
    )
    def scatter_kernel(e_hbm, dst_hbm, zeros_hbm, out_hbm, idx_v, rows_v, acc):
        cid = lax.axis_index("c")
        sid = lax.axis_index("s")
        r0 = sid * rows_per_sub
        pltpu.sync_copy(zeros_hbm.at[pl.ds(r0, rows_per_sub)],
                        acc.at[pl.ds(r0, rows_per_sub)])
        plsc.subcore_barrier()
        for ci in range(n_chunks):
            base = cid * e_per_core + sid * e_per_sub + ci * chunk
            pltpu.sync_copy(dst_hbm.at[pl.ds(base, chunk)], idx_v)
            pltpu.sync_copy(e_hbm.at[pl.ds(base, chunk)], rows_v)
            pltpu.sync_copy(rows_v, acc.at[idx_v], add=True)
        plsc.subcore_barrier()
        pltpu.sync_copy(acc.at[pl.ds(r0, rows_per_sub)],
                        out_hbm.at[pl.ds(cid * N_pad + r0, rows_per_sub)])

    return scatter_kernel


def kernel(efeat, nfeat, edge_index, edge_w1, edge_b1, edge_w2, edge_b2,
           edge_ln_s, edge_ln_b, node_w1, node_b1, node_w2, node_b2,
           node_ln_s, node_ln_b):
    E, D = efeat.shape
    N, _ = nfeat.shape
    L = edge_w1.shape[0]
    H = edge_w1.shape[2]

    tile_e = _pick_tile(E, 1000)
    tile_n = _pick_tile(N, 1000)

    src = edge_index[0]
    dst = edge_index[1]
    n_pad = ((N + 8 * _NS - 1) // (8 * _NS)) * (8 * _NS)
    zeros_nd = jnp.zeros((n_pad, D), jnp.float32)

    proj_call = pl.pallas_call(
        _proj_body,
        grid=(N // tile_n,),
        in_specs=[_row_spec(tile_n, D), _full_spec((D, H)), _full_spec((D, H)),
                  _full_spec((1, H))],
        out_specs=[_row_spec(tile_n, H), _row_spec(tile_n, H)],
        out_shape=[jax.ShapeDtypeStruct((N, H), jnp.float32),
                   jax.ShapeDtypeStruct((N, H), jnp.float32)],
    )

    edge_call = pl.pallas_call(
        _edge_body,
        grid=(E // tile_e,),
        in_specs=[_row_spec(tile_e, D), _row_spec(tile_e, H), _row_spec(tile_e, H),
                  _full_spec((D, H)), _full_spec((H, D)), _full_spec((1, D)),
                  _full_spec((1, D)), _full_spec((1, D))],
        out_specs=_row_spec(tile_e, D),
        out_shape=jax.ShapeDtypeStruct((E, D), jnp.float32),
    )

    node_call = pl.pallas_call(
        _node_body,
        grid=(N // tile_n,),
        in_specs=[_row_spec(tile_n, D), _row_spec(tile_n, D), _row_spec(tile_n, D),
                  _full_spec((D, H)), _full_spec((D, H)), _full_spec((1, H)),
                  _full_spec((H, D)), _full_spec((1, D)), _full_spec((1, D)),
                  _full_spec((1, D))],
        out_specs=_row_spec(tile_n, D),
        out_shape=jax.ShapeDtypeStruct((N, D), jnp.float32),
    )

    gather_call = _make_gather(E, N, H, chunk=400)
    scatter_call = _make_scatter(E, n_pad, D, chunk=200)

    for l in range(L):
        w1 = edge_w1[l]
        p, q = proj_call(nfeat, w1[D:2 * D], w1[2 * D:], edge_b1[l][None, :])
        gs, gd = gather_call(p, q, src, dst)
        efeat = edge_call(efeat, gs, gd, w1[:D], edge_w2[l],
                          edge_b2[l][None, :], edge_ln_s[l][None, :],
                          edge_ln_b[l][None, :])
        partials = scatter_call(efeat, dst, zeros_nd)
        nw1 = node_w1[l]
        nfeat = node_call(partials[:N], partials[n_pad:n_pad + N], nfeat,
                          nw1[:D], nw1[D:], node_b1[l][None, :],
                          node_w2[l], node_b2[l][None, :],
                          node_ln_s[l][None, :], node_ln_b[l][None, :])
    return (efeat, nfeat)


# speedup vs baseline: 3.3208x; 1.1090x over previous
import functools

import jax
import jax.numpy as jnp
from jax import lax
from jax.experimental import pallas as pl
from jax.experimental.pallas import tpu as pltpu
from jax.experimental.pallas import tpu_sc as plsc

EPS = 1e-5


def _proj_body(nf_ref, w1b_ref, w1c_ref, b1_ref, p_ref, q_ref):
    nf = nf_ref[...]
    p_ref[...] = jnp.dot(nf, w1b_ref[...], preferred_element_type=jnp.float32) + b1_ref[...]
    q_ref[...] = jnp.dot(nf, w1c_ref[...], preferred_element_type=jnp.float32)


def _edge_body(e_ref, gs_ref, gd_ref, w1a_ref, w2_ref, b2_ref, s_ref, b_ref, out_ref):
    e = e_ref[...]
    z = jnp.dot(e, w1a_ref[...], preferred_element_type=jnp.float32)
    z = z + gs_ref[...] + gd_ref[...]
    h = z * jax.nn.sigmoid(z)
    m = jnp.dot(h, w2_ref[...], preferred_element_type=jnp.float32) + b2_ref[...]
    mu = jnp.mean(m, axis=-1, keepdims=True)
    c = m - mu
    var = jnp.mean(c * c, axis=-1, keepdims=True)
    out_ref[...] = e + c * jax.lax.rsqrt(var + EPS) * s_ref[...] + b_ref[...]


def _node_body(a0_ref, a1_ref, a2_ref, a3_ref, nf_ref, w1a_ref, w1b_ref, b1_ref,
               w2_ref, b2_ref, s_ref, b_ref, out_ref):
    nf = nf_ref[...]
    agg = (a0_ref[...] + a1_ref[...]) + (a2_ref[...] + a3_ref[...])
    z = jnp.dot(agg, w1a_ref[...], preferred_element_type=jnp.float32)
    z = z + jnp.dot(nf, w1b_ref[...], preferred_element_type=jnp.float32) + b1_ref[...]
    h = z * jax.nn.sigmoid(z)
    m = jnp.dot(h, w2_ref[...], preferred_element_type=jnp.float32) + b2_ref[...]
    mu = jnp.mean(m, axis=-1, keepdims=True)
    c = m - mu
    var = jnp.mean(c * c, axis=-1, keepdims=True)
    out_ref[...] = nf + c * jax.lax.rsqrt(var + EPS) * s_ref[...] + b_ref[...]


def _row_spec(tile, d):
    return pl.BlockSpec((tile, d), lambda i: (i, 0))


def _full_spec(shape):
    return pl.BlockSpec(shape, lambda i: tuple(0 for _ in shape))


def _pick_tile(n, want):
    t = min(want, n)
    while n % t:
        t -= 1
    return t


_SC_MESH = plsc.VectorSubcoreMesh(core_axis_name="c", subcore_axis_name="s")
_NC, _NS = 2, 16
_NW = _NC * _NS


def _make_gather(E, N, H, chunk):
    n_chunks = E // (_NW * chunk)

    @functools.partial(
        pl.kernel,
        out_type=[jax.ShapeDtypeStruct((E, H), jnp.float32),
                  jax.ShapeDtypeStruct((E, H), jnp.float32)],
        mesh=_SC_MESH,
        scratch_types=[
            pltpu.VMEM((chunk,), jnp.int32),
            pltpu.VMEM((chunk,), jnp.int32),
            pltpu.VMEM((chunk, H), jnp.float32),
            pltpu.VMEM((chunk, H), jnp.float32),
            pltpu.SemaphoreType.DMA,
            pltpu.SemaphoreType.DMA,
        ],
    )
    def gather_kernel(p_hbm, q_hbm, src_hbm, dst_hbm, gs_hbm, gd_hbm,
                      idx_s, idx_d, rows_s, rows_d, sem_s, sem_d):
        wid = lax.axis_index("s") * _NC + lax.axis_index("c")
        for ci in range(n_chunks):
            base = (wid * n_chunks + ci) * chunk
            pltpu.sync_copy(src_hbm.at[pl.ds(base, chunk)], idx_s)
            pltpu.sync_copy(dst_hbm.at[pl.ds(base, chunk)], idx_d)
            cp_s = pltpu.async_copy(p_hbm.at[idx_s], rows_s, sem_s)
            cp_d = pltpu.async_copy(q_hbm.at[idx_d], rows_d, sem_d)
            cp_s.wait()
            cp_d.wait()
            pltpu.sync_copy(rows_s, gs_hbm.at[pl.ds(base, chunk)])
            pltpu.sync_copy(rows_d, gd_hbm.at[pl.ds(base, chunk)])

    return gather_kernel


def _make_scatter(E, N_pad, D, chunk):
    e_per_core = E // _NC
    e_per_sub = e_per_core // _NS
    n_chunks = e_per_sub // chunk
    rows_per_sub = N_pad // _NS

    @functools.partial(
        pl.kernel,
        out_type=jax.ShapeDtypeStruct((_NC * N_pad, D), jnp.float32),
        mesh=_SC_MESH,
        scratch_types=[
            pltpu.VMEM((chunk,), jnp.int32),
            pltpu.VMEM((chunk, D), jnp.float32),
            pltpu.VMEM_SHARED((N_pad, D), jnp.float32),
        ],
    )
    def scatter_kernel(e_hbm, dst_hbm, zeros_hbm, out_hbm, idx_v, rows_v, acc):
        cid = lax.axis_index("c")
        sid = lax.axis_index("s")
        r0 = sid * rows_per_sub
        pltpu.sync_copy(zeros_hbm.at[pl.ds(r0, rows_per_sub)],
                        acc.at[pl.ds(r0, rows_per_sub)])
        plsc.subcore_barrier()
        for ci in range(n_chunks):
            base = cid * e_per_core + sid * e_per_sub + ci * chunk
            pltpu.sync_copy(dst_hbm.at[pl.ds(base, chunk)], idx_v)
            pltpu.sync_copy(e_hbm.at[pl.ds(base, chunk)], rows_v)
            pltpu.sync_copy(rows_v, acc.at[idx_v], add=True)
        plsc.subcore_barrier()
        pltpu.sync_copy(acc.at[pl.ds(r0, rows_per_sub)],
                        out_hbm.at[pl.ds(cid * N_pad + r0, rows_per_sub)])

    return scatter_kernel


def kernel(efeat, nfeat, edge_index, edge_w1, edge_b1, edge_w2, edge_b2,
           edge_ln_s, edge_ln_b, node_w1, node_b1, node_w2, node_b2,
           node_ln_s, node_ln_b):
    E, D = efeat.shape
    N, _ = nfeat.shape
    L = edge_w1.shape[0]
    H = edge_w1.shape[2]

    e_half = E // 2
    tile_e = _pick_tile(e_half, 1000)
    tile_n = _pick_tile(N, 1000)

    src = edge_index[0]
    dst = edge_index[1]
    n_pad = ((N + 8 * _NS - 1) // (8 * _NS)) * (8 * _NS)
    zeros_nd = jnp.zeros((n_pad, D), jnp.float32)

    proj_call = pl.pallas_call(
        _proj_body,
        grid=(N // tile_n,),
        in_specs=[_row_spec(tile_n, D), _full_spec((D, H)), _full_spec((D, H)),
                  _full_spec((1, H))],
        out_specs=[_row_spec(tile_n, H), _row_spec(tile_n, H)],
        out_shape=[jax.ShapeDtypeStruct((N, H), jnp.float32),
                   jax.ShapeDtypeStruct((N, H), jnp.float32)],
    )

    edge_call = pl.pallas_call(
        _edge_body,
        grid=(e_half // tile_e,),
        in_specs=[_row_spec(tile_e, D), _row_spec(tile_e, H), _row_spec(tile_e, H),
                  _full_spec((D, H)), _full_spec((H, D)), _full_spec((1, D)),
                  _full_spec((1, D)), _full_spec((1, D))],
        out_specs=_row_spec(tile_e, D),
        out_shape=jax.ShapeDtypeStruct((e_half, D), jnp.float32),
    )

    node_call = pl.pallas_call(
        _node_body,
        grid=(N // tile_n,),
        in_specs=[_row_spec(tile_n, D), _row_spec(tile_n, D), _row_spec(tile_n, D),
                  _row_spec(tile_n, D), _row_spec(tile_n, D),
                  _full_spec((D, H)), _full_spec((D, H)), _full_spec((1, H)),
                  _full_spec((H, D)), _full_spec((1, D)), _full_spec((1, D)),
                  _full_spec((1, D))],
        out_specs=_row_spec(tile_n, D),
        out_shape=jax.ShapeDtypeStruct((N, D), jnp.float32),
    )

    gather_call = _make_gather(e_half, N, H, chunk=200)
    scatter_call = _make_scatter(e_half, n_pad, D, chunk=200)

    src_h = [src[:e_half], src[e_half:]]
    dst_h = [dst[:e_half], dst[e_half:]]
    efeat_h = [efeat[:e_half], efeat[e_half:]]

    for l in range(L):
        w1 = edge_w1[l]
        p, q = proj_call(nfeat, w1[D:2 * D], w1[2 * D:], edge_b1[l][None, :])
        partials = []
        for h in range(2):
            gs, gd = gather_call(p, q, src_h[h], dst_h[h])
            efeat_h[h] = edge_call(efeat_h[h], gs, gd,
                                   w1[:D], edge_w2[l], edge_b2[l][None, :],
                                   edge_ln_s[l][None, :], edge_ln_b[l][None, :])
            partials.append(scatter_call(efeat_h[h], dst_h[h], zeros_nd))
        nw1 = node_w1[l]
        nfeat = node_call(partials[0][:N], partials[0][n_pad:n_pad + N],
                          partials[1][:N], partials[1][n_pad:n_pad + N], nfeat,
                          nw1[:D], nw1[D:], node_b1[l][None, :],
                          node_w2[l], node_b2[l][None, :],
                          node_ln_s[l][None, :], node_ln_b[l][None, :])
    return (jnp.concatenate(efeat_h, axis=0), nfeat)


# speedup vs baseline: 3.4105x; 1.0270x over previous
import functools

import jax
import jax.numpy as jnp
from jax import lax
from jax.experimental import pallas as pl
from jax.experimental.pallas import tpu as pltpu
from jax.experimental.pallas import tpu_sc as plsc

EPS = 1e-5


def _proj_body(nf_ref, w1b_ref, w1c_ref, b1_ref, p_ref, q_ref):
    nf = nf_ref[...]
    p_ref[...] = jnp.dot(nf, w1b_ref[...], preferred_element_type=jnp.float32) + b1_ref[...]
    q_ref[...] = jnp.dot(nf, w1c_ref[...], preferred_element_type=jnp.float32)


def _edge_body(e_ref, gs_ref, gd_ref, w1a_ref, w2_ref, b2_ref, s_ref, b_ref, out_ref):
    e = e_ref[...]
    z = jnp.dot(e, w1a_ref[...], preferred_element_type=jnp.float32)
    z = z + gs_ref[...] + gd_ref[...]
    h = z * jax.nn.sigmoid(z)
    m = jnp.dot(h, w2_ref[...], preferred_element_type=jnp.float32) + b2_ref[...]
    mu = jnp.mean(m, axis=-1, keepdims=True)
    c = m - mu
    var = jnp.mean(c * c, axis=-1, keepdims=True)
    out_ref[...] = e + c * jax.lax.rsqrt(var + EPS) * s_ref[...] + b_ref[...]


def _node_body(a0_ref, a1_ref, a2_ref, a3_ref, nf_ref, w1a_ref, w1b_ref, b1_ref,
               w2_ref, b2_ref, s_ref, b_ref, out_ref):
    nf = nf_ref[...]
    agg = (a0_ref[...] + a1_ref[...]) + (a2_ref[...] + a3_ref[...])
    z = jnp.dot(agg, w1a_ref[...], preferred_element_type=jnp.float32)
    z = z + jnp.dot(nf, w1b_ref[...], preferred_element_type=jnp.float32) + b1_ref[...]
    h = z * jax.nn.sigmoid(z)
    m = jnp.dot(h, w2_ref[...], preferred_element_type=jnp.float32) + b2_ref[...]
    mu = jnp.mean(m, axis=-1, keepdims=True)
    c = m - mu
    var = jnp.mean(c * c, axis=-1, keepdims=True)
    out_ref[...] = nf + c * jax.lax.rsqrt(var + EPS) * s_ref[...] + b_ref[...]


def _row_spec(tile, d):
    return pl.BlockSpec((tile, d), lambda i: (i, 0))


def _full_spec(shape):
    return pl.BlockSpec(shape, lambda i: tuple(0 for _ in shape))


def _pick_tile(n, want):
    t = min(want, n)
    while n % t:
        t -= 1
    return t


_SC_MESH = plsc.VectorSubcoreMesh(core_axis_name="c", subcore_axis_name="s")
_NC, _NS = 2, 16
_NW = _NC * _NS


def _make_gather(E, N, H, chunk):
    per_w = E // _NW
    n_chunks = per_w // chunk

    @functools.partial(
        pl.kernel,
        out_type=[jax.ShapeDtypeStruct((E, H), jnp.float32),
                  jax.ShapeDtypeStruct((E, H), jnp.float32)],
        mesh=_SC_MESH,
        scratch_types=[
            pltpu.VMEM((per_w,), jnp.int32),
            pltpu.VMEM((per_w,), jnp.int32),
            [pltpu.VMEM((chunk, H), jnp.float32) for _ in range(2)],
            [pltpu.VMEM((chunk, H), jnp.float32) for _ in range(2)],
            [pltpu.SemaphoreType.DMA for _ in range(2)],
            [pltpu.SemaphoreType.DMA for _ in range(2)],
        ],
    )
    def gather_kernel(p_hbm, q_hbm, src_hbm, dst_hbm, gs_hbm, gd_hbm,
                      idx_s, idx_d, rows_s, rows_d, sem_g, sem_o):
        wid = lax.axis_index("s") * _NC + lax.axis_index("c")
        tile_base = wid * per_w
        pltpu.sync_copy(src_hbm.at[pl.ds(tile_base, per_w)], idx_s)
        pltpu.sync_copy(dst_hbm.at[pl.ds(tile_base, per_w)], idx_d)

        def start_gather(ci, b):
            lo = pl.ds(ci * chunk, chunk)
            return (pltpu.async_copy(p_hbm.at[idx_s.at[lo]], rows_s[b], sem_g[b]),
                    pltpu.async_copy(q_hbm.at[idx_d.at[lo]], rows_d[b], sem_g[b]))

        def start_out(ci, b):
            lo = pl.ds(tile_base + ci * chunk, chunk)
            return (pltpu.async_copy(rows_s[b], gs_hbm.at[lo], sem_o[b]),
                    pltpu.async_copy(rows_d[b], gd_hbm.at[lo], sem_o[b]))

        gath = [None, None]
        outs = [None, None]
        gath[0] = start_gather(0, 0)
        for ci in range(n_chunks):
            b = ci & 1
            nb = 1 - b
            if ci + 1 < n_chunks:
                if outs[nb] is not None:
                    outs[nb][0].wait()
                    outs[nb][1].wait()
                gath[nb] = start_gather(ci + 1, nb)
            gath[b][0].wait()
            gath[b][1].wait()
            outs[b] = start_out(ci, b)
        for b in range(2):
            if outs[b] is not None:
                outs[b][0].wait()
                outs[b][1].wait()

    return gather_kernel


def _make_scatter(E, N_pad, D, chunk):
    e_per_core = E // _NC
    e_per_sub = e_per_core // _NS
    n_chunks = e_per_sub // chunk
    rows_per_sub = N_pad // _NS

    @functools.partial(
        pl.kernel,
        out_type=jax.ShapeDtypeStruct((_NC * N_pad, D), jnp.float32),
        mesh=_SC_MESH,
        scratch_types=[
            pltpu.VMEM((chunk,), jnp.int32),
            pltpu.VMEM((chunk, D), jnp.float32),
            pltpu.VMEM_SHARED((N_pad, D), jnp.float32),
        ],
    )
    def scatter_kernel(e_hbm, dst_hbm, zeros_hbm, out_hbm, idx_v, rows_v, acc):
        cid = lax.axis_index("c")
        sid = lax.axis_index("s")
        r0 = sid * rows_per_sub
        pltpu.sync_copy(zeros_hbm.at[pl.ds(r0, rows_per_sub)],
                        acc.at[pl.ds(r0, rows_per_sub)])
        plsc.subcore_barrier()
        for ci in range(n_chunks):
            base = cid * e_per_core + sid * e_per_sub + ci * chunk
            pltpu.sync_copy(dst_hbm.at[pl.ds(base, chunk)], idx_v)
            pltpu.sync_copy(e_hbm.at[pl.ds(base, chunk)], rows_v)
            pltpu.sync_copy(rows_v, acc.at[idx_v], add=True)
        plsc.subcore_barrier()
        pltpu.sync_copy(acc.at[pl.ds(r0, rows_per_sub)],
                        out_hbm.at[pl.ds(cid * N_pad + r0, rows_per_sub)])

    return scatter_kernel


def kernel(efeat, nfeat, edge_index, edge_w1, edge_b1, edge_w2, edge_b2,
           edge_ln_s, edge_ln_b, node_w1, node_b1, node_w2, node_b2,
           node_ln_s, node_ln_b):
    E, D = efeat.shape
    N, _ = nfeat.shape
    L = edge_w1.shape[0]
    H = edge_w1.shape[2]

    e_half = E // 2
    tile_e = _pick_tile(e_half, 1000)
    tile_n = _pick_tile(N, 1000)

    src = edge_index[0]
    dst = edge_index[1]
    n_pad = ((N + 8 * _NS - 1) // (8 * _NS)) * (8 * _NS)
    zeros_nd = jnp.zeros((n_pad, D), jnp.float32)

    proj_call = pl.pallas_call(
        _proj_body,
        grid=(N // tile_n,),
        in_specs=[_row_spec(tile_n, D), _full_spec((D, H)), _full_spec((D, H)),
                  _full_spec((1, H))],
        out_specs=[_row_spec(tile_n, H), _row_spec(tile_n, H)],
        out_shape=[jax.ShapeDtypeStruct((N, H), jnp.float32),
                   jax.ShapeDtypeStruct((N, H), jnp.float32)],
    )

    edge_call = pl.pallas_call(
        _edge_body,
        grid=(e_half // tile_e,),
        in_specs=[_row_spec(tile_e, D), _row_spec(tile_e, H), _row_spec(tile_e, H),
                  _full_spec((D, H)), _full_spec((H, D)), _full_spec((1, D)),
                  _full_spec((1, D)), _full_spec((1, D))],
        out_specs=_row_spec(tile_e, D),
        out_shape=jax.ShapeDtypeStruct((e_half, D), jnp.float32),
    )

    node_call = pl.pallas_call(
        _node_body,
        grid=(N // tile_n,),
        in_specs=[_row_spec(tile_n, D), _row_spec(tile_n, D), _row_spec(tile_n, D),
                  _row_spec(tile_n, D), _row_spec(tile_n, D),
                  _full_spec((D, H)), _full_spec((D, H)), _full_spec((1, H)),
                  _full_spec((H, D)), _full_spec((1, D)), _full_spec((1, D)),
                  _full_spec((1, D))],
        out_specs=_row_spec(tile_n, D),
        out_shape=jax.ShapeDtypeStruct((N, D), jnp.float32),
    )

    gather_call = _make_gather(e_half, N, H, chunk=200)
    scatter_call = _make_scatter(e_half, n_pad, D, chunk=200)

    src_h = [src[:e_half], src[e_half:]]
    dst_h = [dst[:e_half], dst[e_half:]]
    efeat_h = [efeat[:e_half], efeat[e_half:]]

    for l in range(L):
        w1 = edge_w1[l]
        p, q = proj_call(nfeat, w1[D:2 * D], w1[2 * D:], edge_b1[l][None, :])
        partials = []
        for h in range(2):
            gs, gd = gather_call(p, q, src_h[h], dst_h[h])
            efeat_h[h] = edge_call(efeat_h[h], gs, gd,
                                   w1[:D], edge_w2[l], edge_b2[l][None, :],
                                   edge_ln_s[l][None, :], edge_ln_b[l][None, :])
            partials.append(scatter_call(efeat_h[h], dst_h[h], zeros_nd))
        nw1 = node_w1[l]
        nfeat = node_call(partials[0][:N], partials[0][n_pad:n_pad + N],
                          partials[1][:N], partials[1][n_pad:n_pad + N], nfeat,
                          nw1[:D], nw1[D:], node_b1[l][None, :],
                          node_w2[l], node_b2[l][None, :],
                          node_ln_s[l][None, :], node_ln_b[l][None, :])
    return (jnp.concatenate(efeat_h, axis=0), nfeat)


# speedup vs baseline: 4.5030x; 1.3204x over previous
import functools

import jax
import jax.numpy as jnp
from jax import lax
from jax.experimental import pallas as pl
from jax.experimental.pallas import tpu as pltpu
from jax.experimental.pallas import tpu_sc as plsc

EPS = 1e-5


def _proj_body(nf_ref, w1b_ref, w1c_ref, b1_ref, p_ref, q_ref):
    nf = nf_ref[...]
    p_ref[...] = jnp.dot(nf, w1b_ref[...], preferred_element_type=jnp.float32) + b1_ref[...]
    q_ref[...] = jnp.dot(nf, w1c_ref[...], preferred_element_type=jnp.float32)


def _edge_body(e_ref, gs_ref, gd_ref, w1a_ref, w2_ref, b2_ref, s_ref, b_ref, out_ref):
    e = e_ref[...]
    z = jnp.dot(e, w1a_ref[...], preferred_element_type=jnp.float32)
    z = z + gs_ref[...] + gd_ref[...]
    h = z * jax.nn.sigmoid(z)
    m = jnp.dot(h, w2_ref[...], preferred_element_type=jnp.float32) + b2_ref[...]
    mu = jnp.mean(m, axis=-1, keepdims=True)
    c = m - mu
    var = jnp.mean(c * c, axis=-1, keepdims=True)
    out_ref[...] = e + c * jax.lax.rsqrt(var + EPS) * s_ref[...] + b_ref[...]


def _node_body(a0_ref, a1_ref, a2_ref, a3_ref, nf_ref, w1a_ref, w1b_ref, b1_ref,
               w2_ref, b2_ref, s_ref, b_ref, out_ref):
    nf = nf_ref[...]
    agg = (a0_ref[...] + a1_ref[...]) + (a2_ref[...] + a3_ref[...])
    z = jnp.dot(agg, w1a_ref[...], preferred_element_type=jnp.float32)
    z = z + jnp.dot(nf, w1b_ref[...], preferred_element_type=jnp.float32) + b1_ref[...]
    h = z * jax.nn.sigmoid(z)
    m = jnp.dot(h, w2_ref[...], preferred_element_type=jnp.float32) + b2_ref[...]
    mu = jnp.mean(m, axis=-1, keepdims=True)
    c = m - mu
    var = jnp.mean(c * c, axis=-1, keepdims=True)
    out_ref[...] = nf + c * jax.lax.rsqrt(var + EPS) * s_ref[...] + b_ref[...]


def _row_spec(tile, d):
    return pl.BlockSpec((tile, d), lambda i: (i, 0))


def _full_spec(shape):
    return pl.BlockSpec(shape, lambda i: tuple(0 for _ in shape))


def _pick_tile(n, want):
    t = min(want, n)
    while n % t:
        t -= 1
    return t


_SC_MESH = plsc.VectorSubcoreMesh(core_axis_name="c", subcore_axis_name="s")
_NC, _NS = 2, 16
_NW = _NC * _NS


def _make_gather(E, N_pad, H, chunk):
    per_w = E // _NS
    n_chunks = per_w // chunk
    rows_per_sub = N_pad // _NS

    @functools.partial(
        pl.kernel,
        out_type=[jax.ShapeDtypeStruct((E, H), jnp.float32),
                  jax.ShapeDtypeStruct((E, H), jnp.float32)],
        mesh=_SC_MESH,
        scratch_types=[
            pltpu.VMEM_SHARED((N_pad, H), jnp.float32),
            pltpu.VMEM((per_w,), jnp.int32),
            [pltpu.VMEM((chunk, H), jnp.float32) for _ in range(2)],
            [pltpu.SemaphoreType.DMA for _ in range(2)],
            [pltpu.SemaphoreType.DMA for _ in range(2)],
        ],
    )
    def gather_kernel(p_hbm, q_hbm, src_hbm, dst_hbm, gs_hbm, gd_hbm,
                      table, idx, rows, sem_g, sem_o):
        cid = lax.axis_index("c")
        sid = lax.axis_index("s")
        r0 = sid * rows_per_sub
        tile_base = sid * per_w

        def run(tab_hbm, idx_hbm, out_hbm):
            pltpu.sync_copy(tab_hbm.at[pl.ds(r0, rows_per_sub)],
                            table.at[pl.ds(r0, rows_per_sub)])
            pltpu.sync_copy(idx_hbm.at[pl.ds(tile_base, per_w)], idx)
            plsc.subcore_barrier()

            def start_gather(ci, b):
                lo = pl.ds(ci * chunk, chunk)
                return pltpu.async_copy(table.at[idx.at[lo]], rows[b], sem_g[b])

            def start_out(ci, b):
                lo = pl.ds(tile_base + ci * chunk, chunk)
                return pltpu.async_copy(rows[b], out_hbm.at[lo], sem_o[b])

            outs = [None, None]
            for ci in range(n_chunks):
                b = ci & 1
                if outs[b] is not None:
                    outs[b].wait()
                g = start_gather(ci, b)
                g.wait()
                outs[b] = start_out(ci, b)
            for b in range(2):
                if outs[b] is not None:
                    outs[b].wait()

        @pl.when(cid == 0)
        def _():
            run(p_hbm, src_hbm, gs_hbm)

        @pl.when(cid == 1)
        def _():
            run(q_hbm, dst_hbm, gd_hbm)

    return gather_kernel


def _make_scatter(E, N_pad, D, chunk):
    e_per_core = E // _NC
    e_per_sub = e_per_core // _NS
    n_chunks = e_per_sub // chunk
    rows_per_sub = N_pad // _NS

    @functools.partial(
        pl.kernel,
        out_type=jax.ShapeDtypeStruct((_NC * N_pad, D), jnp.float32),
        mesh=_SC_MESH,
        scratch_types=[
            pltpu.VMEM((chunk,), jnp.int32),
            pltpu.VMEM((chunk, D), jnp.float32),
            pltpu.VMEM_SHARED((N_pad, D), jnp.float32),
        ],
    )
    def scatter_kernel(e_hbm, dst_hbm, zeros_hbm, out_hbm, idx_v, rows_v, acc):
        cid = lax.axis_index("c")
        sid = lax.axis_index("s")
        r0 = sid * rows_per_sub
        pltpu.sync_copy(zeros_hbm.at[pl.ds(r0, rows_per_sub)],
                        acc.at[pl.ds(r0, rows_per_sub)])
        plsc.subcore_barrier()
        for ci in range(n_chunks):
            base = cid * e_per_core + sid * e_per_sub + ci * chunk
            pltpu.sync_copy(dst_hbm.at[pl.ds(base, chunk)], idx_v)
            pltpu.sync_copy(e_hbm.at[pl.ds(base, chunk)], rows_v)
            pltpu.sync_copy(rows_v, acc.at[idx_v], add=True)
        plsc.subcore_barrier()
        pltpu.sync_copy(acc.at[pl.ds(r0, rows_per_sub)],
                        out_hbm.at[pl.ds(cid * N_pad + r0, rows_per_sub)])

    return scatter_kernel


def kernel(efeat, nfeat, edge_index, edge_w1, edge_b1, edge_w2, edge_b2,
           edge_ln_s, edge_ln_b, node_w1, node_b1, node_w2, node_b2,
           node_ln_s, node_ln_b):
    E, D = efeat.shape
    N, _ = nfeat.shape
    L = edge_w1.shape[0]
    H = edge_w1.shape[2]

    e_half = E // 2
    tile_e = _pick_tile(e_half, 2000)
    tile_n = _pick_tile(N, 1000)
    g_chunk = 128
    e_pad = ((e_half + g_chunk * _NS - 1) // (g_chunk * _NS)) * (g_chunk * _NS)

    src = edge_index[0]
    dst = edge_index[1]
    n_pad = ((N + 255) // 256) * 256
    zeros_nd = jnp.zeros((n_pad, D), jnp.float32)
    tile_p = n_pad // _NS

    proj_call = pl.pallas_call(
        _proj_body,
        grid=(n_pad // tile_p,),
        in_specs=[_row_spec(tile_p, D), _full_spec((D, H)), _full_spec((D, H)),
                  _full_spec((1, H))],
        out_specs=[_row_spec(tile_p, H), _row_spec(tile_p, H)],
        out_shape=[jax.ShapeDtypeStruct((n_pad, H), jnp.float32),
                   jax.ShapeDtypeStruct((n_pad, H), jnp.float32)],
    )

    edge_call = pl.pallas_call(
        _edge_body,
        grid=(e_half // tile_e,),
        in_specs=[_row_spec(tile_e, D),
                  pl.BlockSpec((tile_e, H), lambda i: (i, 0)),
                  pl.BlockSpec((tile_e, H), lambda i: (i, 0)),
                  _full_spec((D, H)), _full_spec((H, D)), _full_spec((1, D)),
                  _full_spec((1, D)), _full_spec((1, D))],
        out_specs=_row_spec(tile_e, D),
        out_shape=jax.ShapeDtypeStruct((e_half, D), jnp.float32),
    )

    node_call = pl.pallas_call(
        _node_body,
        grid=(N // tile_n,),
        in_specs=[_row_spec(tile_n, D), _row_spec(tile_n, D), _row_spec(tile_n, D),
                  _row_spec(tile_n, D), _row_spec(tile_n, D),
                  _full_spec((D, H)), _full_spec((D, H)), _full_spec((1, H)),
                  _full_spec((H, D)), _full_spec((1, D)), _full_spec((1, D)),
                  _full_spec((1, D))],
        out_specs=_row_spec(tile_n, D),
        out_shape=jax.ShapeDtypeStruct((N, D), jnp.float32),
    )

    gather_call = _make_gather(e_pad, n_pad, H, chunk=g_chunk)
    scatter_call = _make_scatter(e_half, n_pad, D, chunk=200)
    pad_n = jnp.zeros((n_pad - N, D), jnp.float32)

    pad_idx = jnp.zeros((e_pad - e_half,), jnp.int32)
    src_h = [jnp.concatenate([src[:e_half], pad_idx]),
             jnp.concatenate([src[e_half:], pad_idx])]
    dst_h = [jnp.concatenate([dst[:e_half], pad_idx]),
             jnp.concatenate([dst[e_half:], pad_idx])]
    dst_s = [dst[:e_half], dst[e_half:]]
    efeat_h = [efeat[:e_half], efeat[e_half:]]

    for l in range(L):
        w1 = edge_w1[l]
        nfeat_pad = jnp.concatenate([nfeat, pad_n], axis=0)
        p, q = proj_call(nfeat_pad, w1[D:2 * D], w1[2 * D:], edge_b1[l][None, :])
        partials = []
        for h in range(2):
            gs, gd = gather_call(p, q, src_h[h], dst_h[h])
            efeat_h[h] = edge_call(efeat_h[h], gs, gd,
                                   w1[:D], edge_w2[l], edge_b2[l][None, :],
                                   edge_ln_s[l][None, :], edge_ln_b[l][None, :])
            partials.append(scatter_call(efeat_h[h], dst_s[h], zeros_nd))
        nw1 = node_w1[l]
        nfeat = node_call(partials[0][:N], partials[0][n_pad:n_pad + N],
                          partials[1][:N], partials[1][n_pad:n_pad + N], nfeat,
                          nw1[:D], nw1[D:], node_b1[l][None, :],
                          node_w2[l], node_b2[l][None, :],
                          node_ln_s[l][None, :], node_ln_b[l][None, :])
    return (jnp.concatenate(efeat_h, axis=0), nfeat)


# speedup vs baseline: 4.6708x; 1.0373x over previous
import functools

import jax
import jax.numpy as jnp
from jax import lax
from jax.experimental import pallas as pl
from jax.experimental.pallas import tpu as pltpu
from jax.experimental.pallas import tpu_sc as plsc

EPS = 1e-5


def _proj_body(nf_ref, w1b_ref, w1c_ref, b1_ref, p_ref, q_ref):
    nf = nf_ref[...]
    p_ref[...] = jnp.dot(nf, w1b_ref[...], preferred_element_type=jnp.float32) + b1_ref[...]
    q_ref[...] = jnp.dot(nf, w1c_ref[...], preferred_element_type=jnp.float32)


def _edge_body(e_ref, gs_ref, gd_ref, w1a_ref, w2_ref, b2_ref, s_ref, b_ref, out_ref):
    e = e_ref[...]
    z = jnp.dot(e, w1a_ref[...], preferred_element_type=jnp.float32)
    z = z + gs_ref[...] + gd_ref[...]
    h = z * jax.nn.sigmoid(z)
    m = jnp.dot(h, w2_ref[...], preferred_element_type=jnp.float32) + b2_ref[...]
    mu = jnp.mean(m, axis=-1, keepdims=True)
    c = m - mu
    var = jnp.mean(c * c, axis=-1, keepdims=True)
    out_ref[...] = e + c * jax.lax.rsqrt(var + EPS) * s_ref[...] + b_ref[...]


def _node_body(a0_ref, a1_ref, a2_ref, a3_ref, nf_ref, w1a_ref, w1b_ref, b1_ref,
               w2_ref, b2_ref, s_ref, b_ref, out_ref):
    nf = nf_ref[...]
    agg = (a0_ref[...] + a1_ref[...]) + (a2_ref[...] + a3_ref[...])
    z = jnp.dot(agg, w1a_ref[...], preferred_element_type=jnp.float32)
    z = z + jnp.dot(nf, w1b_ref[...], preferred_element_type=jnp.float32) + b1_ref[...]
    h = z * jax.nn.sigmoid(z)
    m = jnp.dot(h, w2_ref[...], preferred_element_type=jnp.float32) + b2_ref[...]
    mu = jnp.mean(m, axis=-1, keepdims=True)
    c = m - mu
    var = jnp.mean(c * c, axis=-1, keepdims=True)
    out_ref[...] = nf + c * jax.lax.rsqrt(var + EPS) * s_ref[...] + b_ref[...]


def _row_spec(tile, d):
    return pl.BlockSpec((tile, d), lambda i: (i, 0))


def _full_spec(shape):
    return pl.BlockSpec(shape, lambda i: tuple(0 for _ in shape))


def _pick_tile(n, want):
    t = min(want, n)
    while n % t:
        t -= 1
    return t


_SC_MESH = plsc.VectorSubcoreMesh(core_axis_name="c", subcore_axis_name="s")
_NC, _NS = 2, 16
_NW = _NC * _NS


def _make_gather(E, N_pad, H, chunk):
    per_w = E // _NS
    n_chunks = per_w // chunk
    rows_per_sub = N_pad // _NS

    @functools.partial(
        pl.kernel,
        out_type=[jax.ShapeDtypeStruct((E, H), jnp.float32),
                  jax.ShapeDtypeStruct((E, H), jnp.float32)],
        mesh=_SC_MESH,
        scratch_types=[
            pltpu.VMEM_SHARED((N_pad, H), jnp.float32),
            pltpu.VMEM((per_w,), jnp.int32),
            [pltpu.VMEM((chunk, H), jnp.float32) for _ in range(2)],
            [pltpu.SemaphoreType.DMA for _ in range(2)],
            [pltpu.SemaphoreType.DMA for _ in range(2)],
        ],
    )
    def gather_kernel(p_hbm, q_hbm, src_hbm, dst_hbm, gs_hbm, gd_hbm,
                      table, idx, rows, sem_g, sem_o):
        cid = lax.axis_index("c")
        sid = lax.axis_index("s")
        r0 = sid * rows_per_sub
        tile_base = sid * per_w

        def run(tab_hbm, idx_hbm, out_hbm):
            pltpu.sync_copy(tab_hbm.at[pl.ds(r0, rows_per_sub)],
                            table.at[pl.ds(r0, rows_per_sub)])
            pltpu.sync_copy(idx_hbm.at[pl.ds(tile_base, per_w)], idx)
            plsc.subcore_barrier()

            def start_gather(ci, b):
                lo = pl.ds(ci * chunk, chunk)
                return pltpu.async_copy(table.at[idx.at[lo]], rows[b], sem_g[b])

            def start_out(ci, b):
                lo = pl.ds(tile_base + ci * chunk, chunk)
                return pltpu.async_copy(rows[b], out_hbm.at[lo], sem_o[b])

            outs = [None, None]
            for ci in range(n_chunks):
                b = ci & 1
                if outs[b] is not None:
                    outs[b].wait()
                g = start_gather(ci, b)
                g.wait()
                outs[b] = start_out(ci, b)
            for b in range(2):
                if outs[b] is not None:
                    outs[b].wait()

        @pl.when(cid == 0)
        def _():
            run(p_hbm, src_hbm, gs_hbm)

        @pl.when(cid == 1)
        def _():
            run(q_hbm, dst_hbm, gd_hbm)

    return gather_kernel


def _make_scatter(E, N_pad, D, chunk):
    e_per_core = E // _NC
    e_per_sub = e_per_core // _NS
    n_chunks = e_per_sub // chunk
    rows_per_sub = N_pad // _NS

    @functools.partial(
        pl.kernel,
        out_type=jax.ShapeDtypeStruct((_NC * N_pad, D), jnp.float32),
        mesh=_SC_MESH,
        scratch_types=[
            pltpu.VMEM((chunk,), jnp.int32),
            pltpu.VMEM((chunk, D), jnp.float32),
            pltpu.VMEM_SHARED((N_pad, D), jnp.float32),
        ],
    )
    def scatter_kernel(e_hbm, dst_hbm, zeros_hbm, out_hbm, idx_v, rows_v, acc):
        cid = lax.axis_index("c")
        sid = lax.axis_index("s")
        r0 = sid * rows_per_sub
        pltpu.sync_copy(zeros_hbm.at[pl.ds(r0, rows_per_sub)],
                        acc.at[pl.ds(r0, rows_per_sub)])
        plsc.subcore_barrier()
        for ci in range(n_chunks):
            base = cid * e_per_core + sid * e_per_sub + ci * chunk
            pltpu.sync_copy(dst_hbm.at[pl.ds(base, chunk)], idx_v)
            pltpu.sync_copy(e_hbm.at[pl.ds(base, chunk)], rows_v)
            pltpu.sync_copy(rows_v, acc.at[idx_v], add=True)
        plsc.subcore_barrier()
        pltpu.sync_copy(acc.at[pl.ds(r0, rows_per_sub)],
                        out_hbm.at[pl.ds(cid * N_pad + r0, rows_per_sub)])

    return scatter_kernel


def kernel(efeat, nfeat, edge_index, edge_w1, edge_b1, edge_w2, edge_b2,
           edge_ln_s, edge_ln_b, node_w1, node_b1, node_w2, node_b2,
           node_ln_s, node_ln_b):
    E, D = efeat.shape
    N, _ = nfeat.shape
    L = edge_w1.shape[0]
    H = edge_w1.shape[2]

    e_half = E // 2
    tile_e = _pick_tile(e_half, 2000)
    tile_n = _pick_tile(N, 1000)
    g_chunk = 128
    e_pad = ((e_half + g_chunk * _NS - 1) // (g_chunk * _NS)) * (g_chunk * _NS)

    src = edge_index[0]
    dst = edge_index[1]
    n_pad = ((N + 255) // 256) * 256
    zeros_nd = jnp.zeros((n_pad, D), jnp.float32)
    tile_p = n_pad // _NS

    proj_call = pl.pallas_call(
        _proj_body,
        grid=(n_pad // tile_p,),
        in_specs=[_row_spec(tile_p, D), _full_spec((D, H)), _full_spec((D, H)),
                  _full_spec((1, H))],
        out_specs=[_row_spec(tile_p, H), _row_spec(tile_p, H)],
        out_shape=[jax.ShapeDtypeStruct((n_pad, H), jnp.float32),
                   jax.ShapeDtypeStruct((n_pad, H), jnp.float32)],
    )

    def _make_edge_call(e_off_blocks):
        return pl.pallas_call(
            _edge_body,
            grid=(e_half // tile_e,),
            in_specs=[pl.BlockSpec((tile_e, D),
                                   lambda i: (i + e_off_blocks, 0)),
                      _row_spec(tile_e, H), _row_spec(tile_e, H),
                      _full_spec((D, H)), _full_spec((H, D)), _full_spec((1, D)),
                      _full_spec((1, D)), _full_spec((1, D))],
            out_specs=_row_spec(tile_e, D),
            out_shape=jax.ShapeDtypeStruct((e_half, D), jnp.float32),
        )

    edge_call = _make_edge_call(0)
    edge_call_l0 = [_make_edge_call(0), _make_edge_call(e_half // tile_e)]

    node_call = pl.pallas_call(
        _node_body,
        grid=(N // tile_n,),
        in_specs=[_row_spec(tile_n, D), _row_spec(tile_n, D), _row_spec(tile_n, D),
                  _row_spec(tile_n, D), _row_spec(tile_n, D),
                  _full_spec((D, H)), _full_spec((D, H)), _full_spec((1, H)),
                  _full_spec((H, D)), _full_spec((1, D)), _full_spec((1, D)),
                  _full_spec((1, D))],
        out_specs=_row_spec(tile_n, D),
        out_shape=jax.ShapeDtypeStruct((N, D), jnp.float32),
    )

    gather_call = _make_gather(e_pad, n_pad, H, chunk=g_chunk)
    scatter_call = _make_scatter(e_half, n_pad, D, chunk=200)
    pad_n = jnp.zeros((n_pad - N, D), jnp.float32)

    pad_idx = jnp.zeros((e_pad - e_half,), jnp.int32)
    src_h = [jnp.concatenate([src[:e_half], pad_idx]),
             jnp.concatenate([src[e_half:], pad_idx])]
    dst_h = [jnp.concatenate([dst[:e_half], pad_idx]),
             jnp.concatenate([dst[e_half:], pad_idx])]
    dst_s = [dst[:e_half], dst[e_half:]]
    efeat_h = [efeat, efeat]

    for l in range(L):
        w1 = edge_w1[l]
        nfeat_pad = jnp.concatenate([nfeat, pad_n], axis=0)
        p, q = proj_call(nfeat_pad, w1[D:2 * D], w1[2 * D:], edge_b1[l][None, :])
        partials = []
        for h in range(2):
            gs, gd = gather_call(p, q, src_h[h], dst_h[h])
            ecall = edge_call_l0[h] if l == 0 else edge_call
            efeat_h[h] = ecall(efeat_h[h], gs, gd,
                               w1[:D], edge_w2[l], edge_b2[l][None, :],
                               edge_ln_s[l][None, :], edge_ln_b[l][None, :])
            partials.append(scatter_call(efeat_h[h], dst_s[h], zeros_nd))
        nw1 = node_w1[l]
        nfeat = node_call(partials[0][:N], partials[0][n_pad:n_pad + N],
                          partials[1][:N], partials[1][n_pad:n_pad + N], nfeat,
                          nw1[:D], nw1[D:], node_b1[l][None, :],
                          node_w2[l], node_b2[l][None, :],
                          node_ln_s[l][None, :], node_ln_b[l][None, :])
    return (jnp.concatenate(efeat_h, axis=0), nfeat)


# speedup vs baseline: 4.8473x; 1.0378x over previous
import functools

import jax
import jax.numpy as jnp
from jax import lax
from jax.experimental import pallas as pl
from jax.experimental.pallas import tpu as pltpu
from jax.experimental.pallas import tpu_sc as plsc

EPS = 1e-5


def _proj_body(nf_ref, w1b_ref, w1c_ref, b1_ref, p_ref, q_ref):
    nf = nf_ref[...]
    p_ref[...] = jnp.dot(nf, w1b_ref[...], preferred_element_type=jnp.float32) + b1_ref[...]
    q_ref[...] = jnp.dot(nf, w1c_ref[...], preferred_element_type=jnp.float32)


def _edge_body(e_ref, gs_ref, gd_ref, w1a_ref, w2_ref, b2_ref, s_ref, b_ref, out_ref):
    e = e_ref[...]
    z = jnp.dot(e, w1a_ref[...], preferred_element_type=jnp.float32)
    z = z + gs_ref[...] + gd_ref[...]
    h = z * jax.nn.sigmoid(z)
    m = jnp.dot(h, w2_ref[...], preferred_element_type=jnp.float32) + b2_ref[...]
    mu = jnp.mean(m, axis=-1, keepdims=True)
    c = m - mu
    var = jnp.mean(c * c, axis=-1, keepdims=True)
    out_ref[...] = e + c * jax.lax.rsqrt(var + EPS) * s_ref[...] + b_ref[...]


def _node_body(a0_ref, a1_ref, a2_ref, a3_ref, nf_ref, w1a_ref, w1b_ref, b1_ref,
               w2_ref, b2_ref, s_ref, b_ref, out_ref):
    nf = nf_ref[...]
    agg = (a0_ref[...] + a1_ref[...]) + (a2_ref[...] + a3_ref[...])
    z = jnp.dot(agg, w1a_ref[...], preferred_element_type=jnp.float32)
    z = z + jnp.dot(nf, w1b_ref[...], preferred_element_type=jnp.float32) + b1_ref[...]
    h = z * jax.nn.sigmoid(z)
    m = jnp.dot(h, w2_ref[...], preferred_element_type=jnp.float32) + b2_ref[...]
    mu = jnp.mean(m, axis=-1, keepdims=True)
    c = m - mu
    var = jnp.mean(c * c, axis=-1, keepdims=True)
    out_ref[...] = nf + c * jax.lax.rsqrt(var + EPS) * s_ref[...] + b_ref[...]


def _row_spec(tile, d):
    return pl.BlockSpec((tile, d), lambda i: (i, 0))


def _full_spec(shape):
    return pl.BlockSpec(shape, lambda i: tuple(0 for _ in shape))


def _pick_tile(n, want):
    t = min(want, n)
    while n % t:
        t -= 1
    return t


_SC_MESH = plsc.VectorSubcoreMesh(core_axis_name="c", subcore_axis_name="s")
_NC, _NS = 2, 16
_NW = _NC * _NS


def _make_gather(E, N_pad, H, chunk):
    per_w = E // _NS
    n_chunks = per_w // chunk
    rows_per_sub = N_pad // _NS

    @functools.partial(
        pl.kernel,
        out_type=[jax.ShapeDtypeStruct((E, H), jnp.float32),
                  jax.ShapeDtypeStruct((E, H), jnp.float32)],
        mesh=_SC_MESH,
        scratch_types=[
            pltpu.VMEM_SHARED((N_pad, H), jnp.float32),
            pltpu.VMEM((per_w,), jnp.int32),
            [pltpu.VMEM((chunk, H), jnp.float32) for _ in range(2)],
            [pltpu.SemaphoreType.DMA for _ in range(2)],
            [pltpu.SemaphoreType.DMA for _ in range(2)],
        ],
    )
    def gather_kernel(p_hbm, q_hbm, src_hbm, dst_hbm, gs_hbm, gd_hbm,
                      table, idx, rows, sem_g, sem_o):
        cid = lax.axis_index("c")
        sid = lax.axis_index("s")
        r0 = sid * rows_per_sub
        tile_base = sid * per_w

        def run(tab_hbm, idx_hbm, out_hbm):
            pltpu.sync_copy(tab_hbm.at[pl.ds(r0, rows_per_sub)],
                            table.at[pl.ds(r0, rows_per_sub)])
            pltpu.sync_copy(idx_hbm.at[pl.ds(tile_base, per_w)], idx)
            plsc.subcore_barrier()

            def start_gather(ci, b):
                lo = pl.ds(ci * chunk, chunk)
                return pltpu.async_copy(table.at[idx.at[lo]], rows[b], sem_g[b])

            def start_out(ci, b):
                lo = pl.ds(tile_base + ci * chunk, chunk)
                return pltpu.async_copy(rows[b], out_hbm.at[lo], sem_o[b])

            outs = [None, None]
            for ci in range(n_chunks):
                b = ci & 1
                if outs[b] is not None:
                    outs[b].wait()
                g = start_gather(ci, b)
                g.wait()
                outs[b] = start_out(ci, b)
            for b in range(2):
                if outs[b] is not None:
                    outs[b].wait()

        @pl.when(cid == 0)
        def _():
            run(p_hbm, src_hbm, gs_hbm)

        @pl.when(cid == 1)
        def _():
            run(q_hbm, dst_hbm, gd_hbm)

    return gather_kernel


def _make_scatter(E, N_pad, D, chunk, dump_fill):
    e_per_core = E // _NC
    e_per_sub = e_per_core // _NS
    n_full = e_per_sub // chunk
    tail = e_per_sub - n_full * chunk
    n_chunks = n_full + (1 if tail else 0)
    rows_per_sub = N_pad // _NS

    @functools.partial(
        pl.kernel,
        out_type=jax.ShapeDtypeStruct((_NC * N_pad, D), jnp.float32),
        mesh=_SC_MESH,
        scratch_types=[
            [pltpu.VMEM((chunk,), jnp.int32) for _ in range(2)],
            [pltpu.VMEM((chunk, D), jnp.float32) for _ in range(2)],
            pltpu.VMEM_SHARED((N_pad, D), jnp.float32),
            [pltpu.SemaphoreType.DMA for _ in range(2)],
            [pltpu.SemaphoreType.DMA for _ in range(2)],
        ],
    )
    def scatter_kernel(e_hbm, dst_hbm, zeros_hbm, dump_hbm, out_hbm,
                       idx_v, rows_v, acc, sem_l, sem_a):
        cid = lax.axis_index("c")
        sid = lax.axis_index("s")
        r0 = sid * rows_per_sub
        tile_base = cid * e_per_core + sid * e_per_sub

        def start_load(ci, b):
            if ci < n_full:
                lo = pl.ds(tile_base + ci * chunk, chunk)
                return (pltpu.async_copy(dst_hbm.at[lo], idx_v[b], sem_l[b]),
                        pltpu.async_copy(e_hbm.at[lo], rows_v[b], sem_l[b]))
            lo = pl.ds(tile_base + n_full * chunk, tail)
            return (pltpu.async_copy(dst_hbm.at[lo], idx_v[b].at[pl.ds(0, tail)],
                                     sem_l[b]),
                    pltpu.async_copy(dump_hbm, idx_v[b].at[pl.ds(tail, chunk - tail)],
                                     sem_l[b]),
                    pltpu.async_copy(e_hbm.at[lo], rows_v[b].at[pl.ds(0, tail)],
                                     sem_l[b]))

        loads = [start_load(0, 0), None]
        pltpu.sync_copy(zeros_hbm.at[pl.ds(r0, rows_per_sub)],
                        acc.at[pl.ds(r0, rows_per_sub)])
        plsc.subcore_barrier()
        adds = [None, None]
        for ci in range(n_chunks):
            b = ci & 1
            nb = 1 - b
            if ci + 1 < n_chunks:
                if adds[nb] is not None:
                    adds[nb].wait()
                loads[nb] = start_load(ci + 1, nb)
            for d in loads[b]:
                d.wait()
            adds[b] = pltpu.async_copy(rows_v[b], acc.at[idx_v[b]], sem_a[b],
                                       add=True)
        for b in range(2):
            if adds[b] is not None:
                adds[b].wait()
        plsc.subcore_barrier()
        pltpu.sync_copy(acc.at[pl.ds(r0, rows_per_sub)],
                        out_hbm.at[pl.ds(cid * N_pad + r0, rows_per_sub)])

    return scatter_kernel


def kernel(efeat, nfeat, edge_index, edge_w1, edge_b1, edge_w2, edge_b2,
           edge_ln_s, edge_ln_b, node_w1, node_b1, node_w2, node_b2,
           node_ln_s, node_ln_b):
    E, D = efeat.shape
    N, _ = nfeat.shape
    L = edge_w1.shape[0]
    H = edge_w1.shape[2]

    e_half = E // 2
    tile_e = _pick_tile(e_half, 2000)
    tile_n = _pick_tile(N, 1000)
    g_chunk = 128
    e_pad = ((e_half + g_chunk * _NS - 1) // (g_chunk * _NS)) * (g_chunk * _NS)

    src = edge_index[0]
    dst = edge_index[1]
    n_pad = ((N + 255) // 256) * 256
    zeros_nd = jnp.zeros((n_pad, D), jnp.float32)
    tile_p = n_pad // _NS

    proj_call = pl.pallas_call(
        _proj_body,
        grid=(n_pad // tile_p,),
        in_specs=[_row_spec(tile_p, D), _full_spec((D, H)), _full_spec((D, H)),
                  _full_spec((1, H))],
        out_specs=[_row_spec(tile_p, H), _row_spec(tile_p, H)],
        out_shape=[jax.ShapeDtypeStruct((n_pad, H), jnp.float32),
                   jax.ShapeDtypeStruct((n_pad, H), jnp.float32)],
    )

    def _make_edge_call(e_off_blocks):
        return pl.pallas_call(
            _edge_body,
            grid=(e_half // tile_e,),
            in_specs=[pl.BlockSpec((tile_e, D),
                                   lambda i: (i + e_off_blocks, 0)),
                      _row_spec(tile_e, H), _row_spec(tile_e, H),
                      _full_spec((D, H)), _full_spec((H, D)), _full_spec((1, D)),
                      _full_spec((1, D)), _full_spec((1, D))],
            out_specs=_row_spec(tile_e, D),
            out_shape=jax.ShapeDtypeStruct((e_half, D), jnp.float32),
        )

    edge_call = _make_edge_call(0)
    edge_call_l0 = [_make_edge_call(0), _make_edge_call(e_half // tile_e)]

    node_call = pl.pallas_call(
        _node_body,
        grid=(N // tile_n,),
        in_specs=[_row_spec(tile_n, D), _row_spec(tile_n, D), _row_spec(tile_n, D),
                  _row_spec(tile_n, D), _row_spec(tile_n, D),
                  _full_spec((D, H)), _full_spec((D, H)), _full_spec((1, H)),
                  _full_spec((H, D)), _full_spec((1, D)), _full_spec((1, D)),
                  _full_spec((1, D))],
        out_specs=_row_spec(tile_n, D),
        out_shape=jax.ShapeDtypeStruct((N, D), jnp.float32),
    )

    s_chunk = 184
    s_tail = (e_half // _NW) % s_chunk
    gather_call = _make_gather(e_pad, n_pad, H, chunk=g_chunk)
    scatter_call = _make_scatter(e_half, n_pad, D, chunk=s_chunk, dump_fill=N)
    dump_idx = jnp.full((s_chunk - s_tail,), N, jnp.int32)
    pad_n = jnp.zeros((n_pad - N, D), jnp.float32)

    pad_idx = jnp.zeros((e_pad - e_half,), jnp.int32)
    src_h = [jnp.concatenate([src[:e_half], pad_idx]),
             jnp.concatenate([src[e_half:], pad_idx])]
    dst_h = [jnp.concatenate([dst[:e_half], pad_idx]),
             jnp.concatenate([dst[e_half:], pad_idx])]
    dst_s = [dst[:e_half], dst[e_half:]]
    efeat_h = [efeat, efeat]

    for l in range(L):
        w1 = edge_w1[l]
        nfeat_pad = jnp.concatenate([nfeat, pad_n], axis=0)
        p, q = proj_call(nfeat_pad, w1[D:2 * D], w1[2 * D:], edge_b1[l][None, :])
        partials = []
        for h in range(2):
            gs, gd = gather_call(p, q, src_h[h], dst_h[h])
            ecall = edge_call_l0[h] if l == 0 else edge_call
            efeat_h[h] = ecall(efeat_h[h], gs, gd,
                               w1[:D], edge_w2[l], edge_b2[l][None, :],
                               edge_ln_s[l][None, :], edge_ln_b[l][None, :])
            partials.append(scatter_call(efeat_h[h], dst_s[h], zeros_nd,
                                         dump_idx))
        nw1 = node_w1[l]
        nfeat = node_call(partials[0][:N], partials[0][n_pad:n_pad + N],
                          partials[1][:N], partials[1][n_pad:n_pad + N], nfeat,
                          nw1[:D], nw1[D:], node_b1[l][None, :],
                          node_w2[l], node_b2[l][None, :],
                          node_ln_s[l][None, :], node_ln_b[l][None, :])
    return (jnp.concatenate(efeat_h, axis=0), nfeat)


# speedup vs baseline: 5.3253x; 1.0986x over previous
import functools

import jax
import jax.numpy as jnp
from jax import lax
from jax.experimental import pallas as pl
from jax.experimental.pallas import tpu as pltpu
from jax.experimental.pallas import tpu_sc as plsc

EPS = 1e-5


def _proj_body(nf_ref, w1b_ref, w1c_ref, b1_ref, p_ref, q_ref):
    nf = nf_ref[...]
    p_ref[...] = jnp.dot(nf, w1b_ref[...], preferred_element_type=jnp.float32) + b1_ref[...]
    q_ref[...] = jnp.dot(nf, w1c_ref[...], preferred_element_type=jnp.float32)


def _edge_body(e_ref, gs_ref, gd_ref, w1a_ref, w2_ref, b2_ref, s_ref, b_ref, out_ref):
    e = e_ref[...]
    z = jnp.dot(e, w1a_ref[...], preferred_element_type=jnp.float32)
    z = z + gs_ref[...] + gd_ref[...]
    h = z * jax.nn.sigmoid(z)
    m = jnp.dot(h, w2_ref[...], preferred_element_type=jnp.float32) + b2_ref[...]
    mu = jnp.mean(m, axis=-1, keepdims=True)
    c = m - mu
    var = jnp.mean(c * c, axis=-1, keepdims=True)
    out_ref[...] = e + c * jax.lax.rsqrt(var + EPS) * s_ref[...] + b_ref[...]


def _node_body(a0_ref, a1_ref, a2_ref, a3_ref, nf_ref, w1a_ref, w1b_ref, b1_ref,
               w2_ref, b2_ref, s_ref, b_ref, pw1b_ref, pw1c_ref, pb1_ref,
               out_ref, p_ref, q_ref):
    nf = nf_ref[...]
    agg = (a0_ref[...] + a1_ref[...]) + (a2_ref[...] + a3_ref[...])
    z = jnp.dot(agg, w1a_ref[...], preferred_element_type=jnp.float32)
    z = z + jnp.dot(nf, w1b_ref[...], preferred_element_type=jnp.float32) + b1_ref[...]
    h = z * jax.nn.sigmoid(z)
    m = jnp.dot(h, w2_ref[...], preferred_element_type=jnp.float32) + b2_ref[...]
    mu = jnp.mean(m, axis=-1, keepdims=True)
    c = m - mu
    var = jnp.mean(c * c, axis=-1, keepdims=True)
    n_new = nf + c * jax.lax.rsqrt(var + EPS) * s_ref[...] + b_ref[...]
    out_ref[...] = n_new
    p_ref[...] = jnp.dot(n_new, pw1b_ref[...], preferred_element_type=jnp.float32) + pb1_ref[...]
    q_ref[...] = jnp.dot(n_new, pw1c_ref[...], preferred_element_type=jnp.float32)


def _row_spec(tile, d):
    return pl.BlockSpec((tile, d), lambda i: (i, 0))


def _full_spec(shape):
    return pl.BlockSpec(shape, lambda i: tuple(0 for _ in shape))


def _pick_tile(n, want):
    t = min(want, n)
    while n % t:
        t -= 1
    return t


_SC_MESH = plsc.VectorSubcoreMesh(core_axis_name="c", subcore_axis_name="s")
_NC, _NS = 2, 16
_NW = _NC * _NS


def _make_gather(E, N_pad, H, chunk):
    per_w = E // _NS
    n_chunks = per_w // chunk
    rows_per_sub = N_pad // _NS

    @functools.partial(
        pl.kernel,
        out_type=[jax.ShapeDtypeStruct((E, H), jnp.float32),
                  jax.ShapeDtypeStruct((E, H), jnp.float32)],
        mesh=_SC_MESH,
        scratch_types=[
            pltpu.VMEM_SHARED((N_pad, H), jnp.float32),
            pltpu.VMEM((per_w,), jnp.int32),
            [pltpu.VMEM((chunk, H), jnp.float32) for _ in range(2)],
            [pltpu.SemaphoreType.DMA for _ in range(2)],
            [pltpu.SemaphoreType.DMA for _ in range(2)],
        ],
    )
    def gather_kernel(p_hbm, q_hbm, src_hbm, dst_hbm, gs_hbm, gd_hbm,
                      table, idx, rows, sem_g, sem_o):
        cid = lax.axis_index("c")
        sid = lax.axis_index("s")
        r0 = sid * rows_per_sub
        tile_base = sid * per_w

        def run(tab_hbm, idx_hbm, out_hbm):
            pltpu.sync_copy(tab_hbm.at[pl.ds(r0, rows_per_sub)],
                            table.at[pl.ds(r0, rows_per_sub)])
            pltpu.sync_copy(idx_hbm.at[pl.ds(tile_base, per_w)], idx)
            plsc.subcore_barrier()

            def start_gather(ci, b):
                lo = pl.ds(ci * chunk, chunk)
                return pltpu.async_copy(table.at[idx.at[lo]], rows[b], sem_g[b])

            def start_out(ci, b):
                lo = pl.ds(tile_base + ci * chunk, chunk)
                return pltpu.async_copy(rows[b], out_hbm.at[lo], sem_o[b])

            outs = [None, None]
            for ci in range(n_chunks):
                b = ci & 1
                if outs[b] is not None:
                    outs[b].wait()
                g = start_gather(ci, b)
                g.wait()
                outs[b] = start_out(ci, b)
            for b in range(2):
                if outs[b] is not None:
                    outs[b].wait()

        @pl.when(cid == 0)
        def _():
            run(p_hbm, src_hbm, gs_hbm)

        @pl.when(cid == 1)
        def _():
            run(q_hbm, dst_hbm, gd_hbm)

    return gather_kernel


def _make_scatter(E, N_pad, D, chunk, dump_fill):
    e_per_core = E // _NC
    e_per_sub = e_per_core // _NS
    n_full = e_per_sub // chunk
    tail = e_per_sub - n_full * chunk
    n_chunks = n_full + (1 if tail else 0)
    rows_per_sub = N_pad // _NS

    @functools.partial(
        pl.kernel,
        out_type=jax.ShapeDtypeStruct((_NC * N_pad, D), jnp.float32),
        mesh=_SC_MESH,
        scratch_types=[
            [pltpu.VMEM((chunk,), jnp.int32) for _ in range(2)],
            [pltpu.VMEM((chunk, D), jnp.float32) for _ in range(2)],
            pltpu.VMEM_SHARED((N_pad, D), jnp.float32),
            [pltpu.SemaphoreType.DMA for _ in range(2)],
            [pltpu.SemaphoreType.DMA for _ in range(2)],
        ],
    )
    def scatter_kernel(e_hbm, dst_hbm, zeros_hbm, dump_hbm, out_hbm,
                       idx_v, rows_v, acc, sem_l, sem_a):
        cid = lax.axis_index("c")
        sid = lax.axis_index("s")
        r0 = sid * rows_per_sub
        tile_base = cid * e_per_core + sid * e_per_sub

        def start_load(ci, b):
            if ci < n_full:
                lo = pl.ds(tile_base + ci * chunk, chunk)
                return (pltpu.async_copy(dst_hbm.at[lo], idx_v[b], sem_l[b]),
                        pltpu.async_copy(e_hbm.at[lo], rows_v[b], sem_l[b]))
            lo = pl.ds(tile_base + n_full * chunk, tail)
            return (pltpu.async_copy(dst_hbm.at[lo], idx_v[b].at[pl.ds(0, tail)],
                                     sem_l[b]),
                    pltpu.async_copy(dump_hbm, idx_v[b].at[pl.ds(tail, chunk - tail)],
                                     sem_l[b]),
                    pltpu.async_copy(e_hbm.at[lo], rows_v[b].at[pl.ds(0, tail)],
                                     sem_l[b]))

        loads = [start_load(0, 0), None]
        pltpu.sync_copy(zeros_hbm.at[pl.ds(r0, rows_per_sub)],
                        acc.at[pl.ds(r0, rows_per_sub)])
        plsc.subcore_barrier()
        adds = [None, None]
        for ci in range(n_chunks):
            b = ci & 1
            nb = 1 - b
            if ci + 1 < n_chunks:
                if adds[nb] is not None:
                    adds[nb].wait()
                loads[nb] = start_load(ci + 1, nb)
            for d in loads[b]:
                d.wait()
            adds[b] = pltpu.async_copy(rows_v[b], acc.at[idx_v[b]], sem_a[b],
                                       add=True)
        for b in range(2):
            if adds[b] is not None:
                adds[b].wait()
        plsc.subcore_barrier()
        pltpu.sync_copy(acc.at[pl.ds(r0, rows_per_sub)],
                        out_hbm.at[pl.ds(cid * N_pad + r0, rows_per_sub)])

    return scatter_kernel


def kernel(efeat, nfeat, edge_index, edge_w1, edge_b1, edge_w2, edge_b2,
           edge_ln_s, edge_ln_b, node_w1, node_b1, node_w2, node_b2,
           node_ln_s, node_ln_b):
    E, D = efeat.shape
    N, _ = nfeat.shape
    L = edge_w1.shape[0]
    H = edge_w1.shape[2]

    e_half = E // 2
    tile_e = _pick_tile(e_half, 4000)
    g_chunk = 128
    e_pad = ((e_half + g_chunk * _NS - 1) // (g_chunk * _NS)) * (g_chunk * _NS)

    src = edge_index[0]
    dst = edge_index[1]
    n_pad = ((N + 255) // 256) * 256
    zeros_nd = jnp.zeros((n_pad, D), jnp.float32)
    tile_p = n_pad // _NS

    proj_call = pl.pallas_call(
        _proj_body,
        grid=(n_pad // tile_p,),
        in_specs=[_row_spec(tile_p, D), _full_spec((D, H)), _full_spec((D, H)),
                  _full_spec((1, H))],
        out_specs=[_row_spec(tile_p, H), _row_spec(tile_p, H)],
        out_shape=[jax.ShapeDtypeStruct((n_pad, H), jnp.float32),
                   jax.ShapeDtypeStruct((n_pad, H), jnp.float32)],
    )

    def _make_edge_call(e_off_blocks):
        return pl.pallas_call(
            _edge_body,
            grid=(e_half // tile_e,),
            in_specs=[pl.BlockSpec((tile_e, D),
                                   lambda i: (i + e_off_blocks, 0)),
                      _row_spec(tile_e, H), _row_spec(tile_e, H),
                      _full_spec((D, H)), _full_spec((H, D)), _full_spec((1, D)),
                      _full_spec((1, D)), _full_spec((1, D))],
            out_specs=_row_spec(tile_e, D),
            out_shape=jax.ShapeDtypeStruct((e_half, D), jnp.float32),
        )

    edge_call = _make_edge_call(0)
    edge_call_l0 = [_make_edge_call(0), _make_edge_call(e_half // tile_e)]

    np_blocks = n_pad // tile_p
    node_call = pl.pallas_call(
        _node_body,
        grid=(np_blocks,),
        in_specs=[pl.BlockSpec((tile_p, D), lambda i: (i, 0)),
                  pl.BlockSpec((tile_p, D), lambda i: (i + np_blocks, 0)),
                  pl.BlockSpec((tile_p, D), lambda i: (i, 0)),
                  pl.BlockSpec((tile_p, D), lambda i: (i + np_blocks, 0)),
                  _row_spec(tile_p, D),
                  _full_spec((D, H)), _full_spec((D, H)), _full_spec((1, H)),
                  _full_spec((H, D)), _full_spec((1, D)), _full_spec((1, D)),
                  _full_spec((1, D)),
                  _full_spec((D, H)), _full_spec((D, H)), _full_spec((1, H))],
        out_specs=[_row_spec(tile_p, D), _row_spec(tile_p, H),
                   _row_spec(tile_p, H)],
        out_shape=[jax.ShapeDtypeStruct((n_pad, D), jnp.float32),
                   jax.ShapeDtypeStruct((n_pad, H), jnp.float32),
                   jax.ShapeDtypeStruct((n_pad, H), jnp.float32)],
    )

    s_chunk = 184
    s_tail = (e_half // _NW) % s_chunk
    gather_call = _make_gather(e_pad, n_pad, H, chunk=g_chunk)
    scatter_call = _make_scatter(e_half, n_pad, D, chunk=s_chunk, dump_fill=N)
    dump_idx = jnp.full((s_chunk - s_tail,), N, jnp.int32)
    pad_n = jnp.zeros((n_pad - N, D), jnp.float32)

    pad_idx = jnp.zeros((e_pad - e_half,), jnp.int32)
    src_h = [jnp.concatenate([src[:e_half], pad_idx]),
             jnp.concatenate([src[e_half:], pad_idx])]
    dst_h = [jnp.concatenate([dst[:e_half], pad_idx]),
             jnp.concatenate([dst[e_half:], pad_idx])]
    dst_s = [dst[:e_half], dst[e_half:]]
    efeat_h = [efeat, efeat]

    nfeat_pad = jnp.concatenate([nfeat, pad_n], axis=0)
    w1_0 = edge_w1[0]
    p, q = proj_call(nfeat_pad, w1_0[D:2 * D], w1_0[2 * D:],
                     edge_b1[0][None, :])

    for l in range(L):
        w1 = edge_w1[l]
        partials = []
        for h in range(2):
            gs, gd = gather_call(p, q, src_h[h], dst_h[h])
            ecall = edge_call_l0[h] if l == 0 else edge_call
            efeat_h[h] = ecall(efeat_h[h], gs, gd,
                               w1[:D], edge_w2[l], edge_b2[l][None, :],
                               edge_ln_s[l][None, :], edge_ln_b[l][None, :])
            partials.append(scatter_call(efeat_h[h], dst_s[h], zeros_nd,
                                         dump_idx))
        nw1 = node_w1[l]
        w1n = edge_w1[(l + 1) % L]
        nfeat_pad, p, q = node_call(
            partials[0], partials[0], partials[1], partials[1], nfeat_pad,
            nw1[:D], nw1[D:], node_b1[l][None, :],
            node_w2[l], node_b2[l][None, :],
            node_ln_s[l][None, :], node_ln_b[l][None, :],
            w1n[D:2 * D], w1n[2 * D:], edge_b1[(l + 1) % L][None, :])
    return (jnp.concatenate(efeat_h, axis=0), nfeat_pad[:N])
